# baseline (speedup 1.0000x reference)
import jax
import jax.numpy as jnp
from jax import lax
from jax.experimental import pallas as pl
from jax.experimental.pallas import tpu as pltpu
from jax.experimental.pallas import tpu_sc as plsc

N = 10000
E = 320000
D = 128
G = 64
NCLS = 10

NC = 2
NS = 16
NW = NC * NS

P = 10240
EP = 327680
EW = EP // NW
K = 128
NCH = EW // K
NCHT = EP // K
F0 = 120
F1 = NCH * 2 - F0
RT = P // NS
RC = RT // K

BLK = 512
NBLK = P // BLK


def _deg_body(dst2, zvec, out, dstbuf, degbuf):
    cid = lax.axis_index("c")
    sid = lax.axis_index("s")
    wid = cid * NS + sid
    pltpu.sync_copy(dst2.at[pl.ds(wid * NCH, NCH)], dstbuf)
    pltpu.sync_copy(zvec, degbuf)
    ones = jnp.ones((16,), jnp.float32)

    def step(i, carry):
        r = i // 8
        j = i % 8
        idx = dstbuf[r, pl.ds(pl.multiple_of(j * 16, 16), 16)]
        plsc.addupdate_scatter(degbuf, [idx], ones)
        return carry

    lax.fori_loop(0, NCH * 8, step, 0)
    pltpu.sync_copy(degbuf, out.at[wid])


def _deg_call(dst2, zvec):
    mesh = plsc.VectorSubcoreMesh(core_axis_name="c", subcore_axis_name="s")
    return pl.kernel(
        _deg_body,
        out_type=jax.ShapeDtypeStruct((NW, P), jnp.float32),
        mesh=mesh,
        scratch_types=[
            pltpu.VMEM((NCH, K), jnp.int32),
            pltpu.VMEM((P,), jnp.float32),
        ],
        compiler_params=pltpu.CompilerParams(needs_layout_passes=False),
    )(dst2, zvec)


NB = 2
NI = 4


def _agg_body(g, src1, dst2, zmat, out, srcbuf, acc, r0, r1, d0, d1, d2, d3,
              gs0, gs1, ss0, ss1, is0, is1, is2, is3):
    cid = lax.axis_index("c")
    sid = lax.axis_index("s")
    rows = [r0, r1]
    didx = [d0, d1, d2, d3]
    gsem = [gs0, gs1]
    ssem = [ss0, ss1]
    isem = [is0, is1, is2, is3]

    coff = jnp.where(cid == 0, sid * F0, NS * F0 + sid * F1)
    nch = jnp.where(cid == 0, F0, F1)

    pltpu.sync_copy(zmat, r0)
    for k in range(RC):
        pltpu.sync_copy(r0, acc.at[pl.ds(sid * RT + k * K, K)])

    @pl.when(cid == 0)
    def _():
        pltpu.sync_copy(src1.at[pl.ds(coff * K, F0 * K)], srcbuf.at[pl.ds(0, F0 * K)])

    @pl.when(cid == 1)
    def _():
        pltpu.sync_copy(src1.at[pl.ds(coff * K, F1 * K)], srcbuf.at[pl.ds(0, F1 * K)])

    plsc.subcore_barrier()

    for q in range(NI):
        pltpu.async_copy(dst2.at[coff + q], didx[q], isem[q])
    for b in range(NB):
        pltpu.async_copy(g.at[srcbuf.at[pl.ds(b * K, K)]], rows[b], gsem[b])

    def step(t, carry):
        for cc in range(NI):
            c = t * NI + cc
            b = cc % NB
            q = cc
            pltpu.make_async_copy(
                g.at[srcbuf.at[pl.ds(c * K, K)]], rows[b], gsem[b]
            ).wait()
            pltpu.make_async_copy(dst2.at[coff + c], didx[q], isem[q]).wait()
            pltpu.async_copy(rows[b], acc.at[didx[q]], ssem[b], add=True)
            pltpu.make_async_copy(rows[b], acc.at[didx[q]], ssem[b]).wait()

            @pl.when(c + NI < nch)
            def _():
                pltpu.async_copy(dst2.at[coff + c + NI], didx[q], isem[q])

            @pl.when(c + NB < nch)
            def _():
                pltpu.async_copy(
                    g.at[srcbuf.at[pl.ds((c + NB) * K, K)]], rows[b], gsem[b]
                )

        return carry

    lax.fori_loop(0, nch // NI, step, 0)
    plsc.subcore_barrier()
    for k in range(RC):
        base = sid * RT + k * K
        pltpu.sync_copy(acc.at[pl.ds(base, K)], r0)
        pltpu.sync_copy(r0, out.at[cid, pl.ds(base, K)])


def _agg_call(g, src1, dst2, zmat):
    mesh = plsc.VectorSubcoreMesh(core_axis_name="c", subcore_axis_name="s")
    return pl.kernel(
        _agg_body,
        out_type=jax.ShapeDtypeStruct((NC, P, D), jnp.float32),
        mesh=mesh,
        scratch_types=[
            pltpu.VMEM((F0 * K,), jnp.int32),
            pltpu.VMEM_SHARED((P, D), jnp.float32),
        ]
        + [pltpu.VMEM((K, D), jnp.float32)] * NB
        + [pltpu.VMEM((K,), jnp.int32)] * NI
        + [pltpu.SemaphoreType.DMA] * (NB + NB + NI),
    )(g, src1, dst2, zmat)


def _mm_body(x_ref, w_ref, o_ref):
    o_ref[...] = jnp.dot(x_ref[...], w_ref[...], preferred_element_type=jnp.float32)


def _mm_call(x, w):
    return pl.pallas_call(
        _mm_body,
        grid=(NBLK,),
        in_specs=[
            pl.BlockSpec((BLK, D), lambda i: (i, 0)),
            pl.BlockSpec((D, D), lambda i: (0, 0)),
        ],
        out_specs=pl.BlockSpec((BLK, D), lambda i: (i, 0)),
        out_shape=jax.ShapeDtypeStruct((P, D), jnp.float32),
    )(x, w)


def _scale_body(degt_ref, xw_ref, g_ref, dinv_ref):
    deg = jnp.sum(degt_ref[...], axis=1, keepdims=True) + 1.0
    dv = lax.rsqrt(deg)
    g_ref[...] = xw_ref[...] * dv
    dinv_ref[...] = jnp.broadcast_to(dv, (BLK, D))


def _scale_call(degt, xw):
    return pl.pallas_call(
        _scale_body,
        grid=(NBLK,),
        in_specs=[
            pl.BlockSpec((BLK, NW), lambda i: (i, 0)),
            pl.BlockSpec((BLK, D), lambda i: (i, 0)),
        ],
        out_specs=[
            pl.BlockSpec((BLK, D), lambda i: (i, 0)),
            pl.BlockSpec((BLK, D), lambda i: (i, 0)),
        ],
        out_shape=[
            jax.ShapeDtypeStruct((P, D), jnp.float32),
            jax.ShapeDtypeStruct((P, D), jnp.float32),
        ],
    )(degt, xw)


def _comb_mm_body(p0_ref, p1_ref, g_ref, dinv_ref, b_ref, w_ref, o_ref):
    h = dinv_ref[...] * (p0_ref[...] + p1_ref[...] + g_ref[...]) + b_ref[...]
    h = jnp.maximum(h, 0.0)
    o_ref[...] = (
        jnp.dot(h, w_ref[...], preferred_element_type=jnp.float32) * dinv_ref[...]
    )


def _comb_mm_call(p0, p1, g, dinv, b, w):
    return pl.pallas_call(
        _comb_mm_body,
        grid=(NBLK,),
        in_specs=[
            pl.BlockSpec((BLK, D), lambda i: (i, 0)),
            pl.BlockSpec((BLK, D), lambda i: (i, 0)),
            pl.BlockSpec((BLK, D), lambda i: (i, 0)),
            pl.BlockSpec((BLK, D), lambda i: (i, 0)),
            pl.BlockSpec((1, D), lambda i: (0, 0)),
            pl.BlockSpec((D, D), lambda i: (0, 0)),
        ],
        out_specs=pl.BlockSpec((BLK, D), lambda i: (i, 0)),
        out_shape=jax.ShapeDtypeStruct((P, D), jnp.float32),
    )(p0, p1, g, dinv, b, w)


def _final_body(q0_ref, q1_ref, g_ref, dinv_ref, b_ref, bat_ref, wf_ref, bf_ref,
                o_ref, sums):
    i = pl.program_id(0)

    @pl.when(i == 0)
    def _():
        sums[...] = jnp.zeros_like(sums)

    h = dinv_ref[...] * (q0_ref[...] + q1_ref[...] + g_ref[...]) + b_ref[...]
    h = jnp.maximum(h, 0.0)
    hx = jnp.concatenate([h, jnp.ones((BLK, D), jnp.float32)], axis=1)
    bt = jnp.broadcast_to(bat_ref[...], (G, BLK))
    rows = lax.broadcasted_iota(jnp.int32, (G, BLK), 0)
    oht = (bt == rows).astype(jnp.float32)
    sums[...] += jnp.dot(oht, hx, preferred_element_type=jnp.float32)

    @pl.when(i == NBLK - 1)
    def _():
        s = sums[...]
        pooled = s[:, :D] / jnp.maximum(s[:, D:], 1.0)
        logits = (
            jnp.dot(pooled, wf_ref[...], preferred_element_type=jnp.float32)
            + bf_ref[...]
        )
        m = jnp.max(logits, axis=1, keepdims=True)
        e = jnp.exp(logits - m)
        o_ref[...] = (e / jnp.sum(e, axis=1, keepdims=True))[:, :NCLS]


def _final_call(q0, q1, g, dinv, b, bat, wf, bf):
    return pl.pallas_call(
        _final_body,
        grid=(NBLK,),
        in_specs=[
            pl.BlockSpec((BLK, D), lambda i: (i, 0)),
            pl.BlockSpec((BLK, D), lambda i: (i, 0)),
            pl.BlockSpec((BLK, D), lambda i: (i, 0)),
            pl.BlockSpec((BLK, D), lambda i: (i, 0)),
            pl.BlockSpec((1, D), lambda i: (0, 0)),
            pl.BlockSpec((1, BLK), lambda i: (0, i)),
            pl.BlockSpec((D, D), lambda i: (0, 0)),
            pl.BlockSpec((1, D), lambda i: (0, 0)),
        ],
        out_specs=pl.BlockSpec((G, NCLS), lambda i: (0, 0)),
        out_shape=jax.ShapeDtypeStruct((G, NCLS), jnp.float32),
        scratch_shapes=[pltpu.VMEM((G, 2 * D), jnp.float32)],
    )(q0, q1, g, dinv, b, bat, wf, bf)


def kernel(x, edge_index, batch, W1, b1, W2, b2, Wf, bf):
    src = edge_index[0].astype(jnp.int32)
    dst = edge_index[1].astype(jnp.int32)
    npad = EP - E
    src1 = jnp.concatenate([src, jnp.full((npad,), N, jnp.int32)])
    dst2 = jnp.concatenate([dst, jnp.full((npad,), N, jnp.int32)]).reshape(NCHT, K)
    x_p = jnp.pad(x, ((0, P - N), (0, 0)))
    bat1 = jnp.concatenate(
        [batch.astype(jnp.int32), jnp.full((P - N,), G, jnp.int32)]
    ).reshape(1, P)
    zmat = jnp.zeros((K, D), jnp.float32)
    zvec = jnp.zeros((P,), jnp.float32)
    b1r = b1.reshape(1, D)
    b2r = b2.reshape(1, D)
    wfp = jnp.pad(Wf, ((0, 0), (0, D - NCLS)))
    bfp = jnp.concatenate([bf, jnp.full((D - NCLS,), -1e30, jnp.float32)]).reshape(1, D)

    degp = _deg_call(dst2, zvec)
    xw = _mm_call(x_p, W1)
    g1, dinv = _scale_call(degp.T, xw)
    agg1 = _agg_call(g1, src1, dst2, zmat)
    g2 = _comb_mm_call(agg1[0], agg1[1], g1, dinv, b1r, W2)
    agg2 = _agg_call(g2, src1, dst2, zmat)
    return _final_call(agg2[0], agg2[1], g2, dinv, b2r, bat1, wfp, bfp)


# speedup vs baseline: 3.1046x; 3.1046x over previous
import jax
import jax.numpy as jnp
from jax import lax
from jax.experimental import pallas as pl
from jax.experimental.pallas import tpu as pltpu
from jax.experimental.pallas import tpu_sc as plsc

N = 10000
E = 320000
D = 128
G = 64
NCLS = 10

NC = 2
NS = 16
NW = NC * NS

P = 10240
EP = 327680
EW = EP // NW
K = 128
NCH = EW // K
NCHT = EP // K
F0 = 80
F1 = NCH * 2 - F0
RT = P // NS
RC = RT // K

BLK = 512
NBLK = P // BLK


def _deg_body(dst2, zvec, out, dstbuf, degbuf):
    cid = lax.axis_index("c")
    sid = lax.axis_index("s")
    wid = cid * NS + sid
    pltpu.sync_copy(dst2.at[pl.ds(wid * NCH, NCH)], dstbuf)
    pltpu.sync_copy(zvec, degbuf)
    ones = jnp.ones((16,), jnp.float32)

    def step(i, carry):
        r = i // 8
        j = i % 8
        idx = dstbuf[r, pl.ds(pl.multiple_of(j * 16, 16), 16)]
        plsc.addupdate_scatter(degbuf, [idx], ones)
        return carry

    lax.fori_loop(0, NCH * 8, step, 0)
    pltpu.sync_copy(degbuf, out.at[wid])


def _deg_call(dst2, zvec):
    mesh = plsc.VectorSubcoreMesh(core_axis_name="c", subcore_axis_name="s")
    return pl.kernel(
        _deg_body,
        out_type=jax.ShapeDtypeStruct((NW, P), jnp.float32),
        mesh=mesh,
        scratch_types=[
            pltpu.VMEM((NCH, K), jnp.int32),
            pltpu.VMEM((P,), jnp.float32),
        ],
        compiler_params=pltpu.CompilerParams(needs_layout_passes=False),
    )(dst2, zvec)


NB = 2
NI = 4


def _agg_body(g, src1, dst2, zmat, out, srcbuf, acc, r0, r1, d0, d1, d2, d3,
              gs0, gs1, ss0, ss1, is0, is1, is2, is3):
    cid = lax.axis_index("c")
    sid = lax.axis_index("s")
    rows = [r0, r1]
    didx = [d0, d1, d2, d3]
    gsem = [gs0, gs1]
    ssem = [ss0, ss1]
    isem = [is0, is1, is2, is3]

    coff = jnp.where(cid == 0, sid * F0, NS * F0 + sid * F1)
    nch = jnp.where(cid == 0, F0, F1)

    pltpu.sync_copy(zmat, r0)
    for k in range(RC):
        pltpu.sync_copy(r0, acc.at[pl.ds(sid * RT + k * K, K)])

    @pl.when(cid == 0)
    def _():
        pltpu.sync_copy(src1.at[pl.ds(coff * K, F0 * K)], srcbuf.at[pl.ds(0, F0 * K)])

    @pl.when(cid == 1)
    def _():
        pltpu.sync_copy(src1.at[pl.ds(coff * K, F1 * K)], srcbuf.at[pl.ds(0, F1 * K)])

    plsc.subcore_barrier()

    for q in range(NI):
        pltpu.async_copy(dst2.at[coff + q], didx[q], isem[q])
    for b in range(NB):
        pltpu.async_copy(g.at[srcbuf.at[pl.ds(b * K, K)]], rows[b], gsem[b])

    def step(t, carry):
        for cc in range(NI):
            c = t * NI + cc
            b = cc % NB
            q = cc
            pltpu.make_async_copy(
                g.at[srcbuf.at[pl.ds(c * K, K)]], rows[b], gsem[b]
            ).wait()
            pltpu.make_async_copy(dst2.at[coff + c], didx[q], isem[q]).wait()
            pltpu.async_copy(rows[b], acc.at[didx[q]], ssem[b], add=True)
            pltpu.make_async_copy(rows[b], acc.at[didx[q]], ssem[b]).wait()

            @pl.when(c + NI < nch)
            def _():
                pltpu.async_copy(dst2.at[coff + c + NI], didx[q], isem[q])

            @pl.when(c + NB < nch)
            def _():
                pltpu.async_copy(
                    g.at[srcbuf.at[pl.ds((c + NB) * K, K)]], rows[b], gsem[b]
                )

        return carry

    lax.fori_loop(0, nch // NI, step, 0)
    plsc.subcore_barrier()
    for k in range(RC):
        base = sid * RT + k * K
        pltpu.sync_copy(acc.at[pl.ds(base, K)], r0)
        pltpu.sync_copy(r0, out.at[cid, pl.ds(base, K)])


def _agg_call(g, src1, dst2, zmat):
    mesh = plsc.VectorSubcoreMesh(core_axis_name="c", subcore_axis_name="s")
    return pl.kernel(
        _agg_body,
        out_type=jax.ShapeDtypeStruct((NC, P, D), jnp.float32),
        mesh=mesh,
        scratch_types=[
            pltpu.VMEM((F0 * K,), jnp.int32),
            pltpu.VMEM_SHARED((P, D), jnp.float32),
        ]
        + [pltpu.VMEM((K, D), jnp.float32)] * NB
        + [pltpu.VMEM((K,), jnp.int32)] * NI
        + [pltpu.SemaphoreType.DMA] * (NB + NB + NI),
    )(g, src1, dst2, zmat)


def _mm_body(x_ref, w_ref, o_ref):
    o_ref[...] = jnp.dot(x_ref[...], w_ref[...], preferred_element_type=jnp.float32)


def _mm_call(x, w):
    return pl.pallas_call(
        _mm_body,
        grid=(NBLK,),
        in_specs=[
            pl.BlockSpec((BLK, D), lambda i: (i, 0)),
            pl.BlockSpec((D, D), lambda i: (0, 0)),
        ],
        out_specs=pl.BlockSpec((BLK, D), lambda i: (i, 0)),
        out_shape=jax.ShapeDtypeStruct((P, D), jnp.float32),
    )(x, w)


def _scale_body(degt_ref, xw_ref, g_ref, dinv_ref):
    deg = jnp.sum(degt_ref[...], axis=1, keepdims=True) + 1.0
    dv = lax.rsqrt(deg)
    g_ref[...] = xw_ref[...] * dv
    dinv_ref[...] = jnp.broadcast_to(dv, (BLK, D))


def _scale_call(degt, xw):
    return pl.pallas_call(
        _scale_body,
        grid=(NBLK,),
        in_specs=[
            pl.BlockSpec((BLK, NW), lambda i: (i, 0)),
            pl.BlockSpec((BLK, D), lambda i: (i, 0)),
        ],
        out_specs=[
            pl.BlockSpec((BLK, D), lambda i: (i, 0)),
            pl.BlockSpec((BLK, D), lambda i: (i, 0)),
        ],
        out_shape=[
            jax.ShapeDtypeStruct((P, D), jnp.float32),
            jax.ShapeDtypeStruct((P, D), jnp.float32),
        ],
    )(degt, xw)


def _comb_mm_body(p0_ref, p1_ref, g_ref, dinv_ref, b_ref, w_ref, o_ref):
    h = dinv_ref[...] * (p0_ref[...] + p1_ref[...] + g_ref[...]) + b_ref[...]
    h = jnp.maximum(h, 0.0)
    o_ref[...] = (
        jnp.dot(h, w_ref[...], preferred_element_type=jnp.float32) * dinv_ref[...]
    )


def _comb_mm_call(p0, p1, g, dinv, b, w):
    return pl.pallas_call(
        _comb_mm_body,
        grid=(NBLK,),
        in_specs=[
            pl.BlockSpec((BLK, D), lambda i: (i, 0)),
            pl.BlockSpec((BLK, D), lambda i: (i, 0)),
            pl.BlockSpec((BLK, D), lambda i: (i, 0)),
            pl.BlockSpec((BLK, D), lambda i: (i, 0)),
            pl.BlockSpec((1, D), lambda i: (0, 0)),
            pl.BlockSpec((D, D), lambda i: (0, 0)),
        ],
        out_specs=pl.BlockSpec((BLK, D), lambda i: (i, 0)),
        out_shape=jax.ShapeDtypeStruct((P, D), jnp.float32),
    )(p0, p1, g, dinv, b, w)


def _final_body(q0_ref, q1_ref, g_ref, dinv_ref, b_ref, bat_ref, wf_ref, bf_ref,
                o_ref, sums):
    i = pl.program_id(0)

    @pl.when(i == 0)
    def _():
        sums[...] = jnp.zeros_like(sums)

    h = dinv_ref[...] * (q0_ref[...] + q1_ref[...] + g_ref[...]) + b_ref[...]
    h = jnp.maximum(h, 0.0)
    hx = jnp.concatenate([h, jnp.ones((BLK, D), jnp.float32)], axis=1)
    bt = jnp.broadcast_to(bat_ref[...], (G, BLK))
    rows = lax.broadcasted_iota(jnp.int32, (G, BLK), 0)
    oht = (bt == rows).astype(jnp.float32)
    sums[...] += jnp.dot(oht, hx, preferred_element_type=jnp.float32)

    @pl.when(i == NBLK - 1)
    def _():
        s = sums[...]
        pooled = s[:, :D] / jnp.maximum(s[:, D:], 1.0)
        logits = (
            jnp.dot(pooled, wf_ref[...], preferred_element_type=jnp.float32)
            + bf_ref[...]
        )
        m = jnp.max(logits, axis=1, keepdims=True)
        e = jnp.exp(logits - m)
        o_ref[...] = (e / jnp.sum(e, axis=1, keepdims=True))[:, :NCLS]


def _final_call(q0, q1, g, dinv, b, bat, wf, bf):
    return pl.pallas_call(
        _final_body,
        grid=(NBLK,),
        in_specs=[
            pl.BlockSpec((BLK, D), lambda i: (i, 0)),
            pl.BlockSpec((BLK, D), lambda i: (i, 0)),
            pl.BlockSpec((BLK, D), lambda i: (i, 0)),
            pl.BlockSpec((BLK, D), lambda i: (i, 0)),
            pl.BlockSpec((1, D), lambda i: (0, 0)),
            pl.BlockSpec((1, BLK), lambda i: (0, i)),
            pl.BlockSpec((D, D), lambda i: (0, 0)),
            pl.BlockSpec((1, D), lambda i: (0, 0)),
        ],
        out_specs=pl.BlockSpec((G, NCLS), lambda i: (0, 0)),
        out_shape=jax.ShapeDtypeStruct((G, NCLS), jnp.float32),
        scratch_shapes=[pltpu.VMEM((G, 2 * D), jnp.float32)],
    )(q0, q1, g, dinv, b, bat, wf, bf)


def kernel(x, edge_index, batch, W1, b1, W2, b2, Wf, bf):
    src = edge_index[0].astype(jnp.int32)
    dst = edge_index[1].astype(jnp.int32)
    npad = EP - E
    pad_ids = N + jnp.arange(npad, dtype=jnp.int32) % (P - N)
    src1 = jnp.concatenate([src, pad_ids])
    dst2 = jnp.concatenate([dst, pad_ids]).reshape(NCHT, K)
    x_p = jnp.pad(x, ((0, P - N), (0, 0)))
    bat1 = jnp.concatenate(
        [batch.astype(jnp.int32), jnp.full((P - N,), G, jnp.int32)]
    ).reshape(1, P)
    zmat = jnp.zeros((K, D), jnp.float32)
    zvec = jnp.zeros((P,), jnp.float32)
    b1r = b1.reshape(1, D)
    b2r = b2.reshape(1, D)
    wfp = jnp.pad(Wf, ((0, 0), (0, D - NCLS)))
    bfp = jnp.concatenate([bf, jnp.full((D - NCLS,), -1e30, jnp.float32)]).reshape(1, D)

    degp = _deg_call(dst2, zvec)
    xw = _mm_call(x_p, W1)
    g1, dinv = _scale_call(degp.T, xw)
    agg1 = _agg_call(g1, src1, dst2, zmat)
    g2 = _comb_mm_call(agg1[0], agg1[1], g1, dinv, b1r, W2)
    agg2 = _agg_call(g2, src1, dst2, zmat)
    return _final_call(agg2[0], agg2[1], g2, dinv, b2r, bat1, wfp, bfp)


# speedup vs baseline: 3.2530x; 1.0478x over previous
import jax
import jax.numpy as jnp
from jax import lax
from jax.experimental import pallas as pl
from jax.experimental.pallas import tpu as pltpu
from jax.experimental.pallas import tpu_sc as plsc

N = 10000
E = 320000
D = 128
G = 64
NCLS = 10

NC = 2
NS = 16
NW = NC * NS

P = 10240
EP = 327680
EW = EP // NW
K = 128
NCH = EW // K
NCHT = EP // K
F0 = 80
F1 = NCH * 2 - F0
RT = P // NS
RC = RT // K

BLK = 512
NBLK = P // BLK


def _deg_body(dst2, zvec, out, dstbuf, degbuf):
    cid = lax.axis_index("c")
    sid = lax.axis_index("s")
    wid = cid * NS + sid
    pltpu.sync_copy(dst2.at[pl.ds(wid * NCH, NCH)], dstbuf)
    pltpu.sync_copy(zvec, degbuf)
    ones = jnp.ones((16,), jnp.float32)

    def step(i, carry):
        r = i // 8
        j = i % 8
        idx = dstbuf[r, pl.ds(pl.multiple_of(j * 16, 16), 16)]
        plsc.addupdate_scatter(degbuf, [idx], ones)
        return carry

    lax.fori_loop(0, NCH * 8, step, 0)
    pltpu.sync_copy(degbuf, out.at[wid])


def _deg_call(dst2, zvec):
    mesh = plsc.VectorSubcoreMesh(core_axis_name="c", subcore_axis_name="s")
    return pl.kernel(
        _deg_body,
        out_type=jax.ShapeDtypeStruct((NW, P), jnp.float32),
        mesh=mesh,
        scratch_types=[
            pltpu.VMEM((NCH, K), jnp.int32),
            pltpu.VMEM((P,), jnp.float32),
        ],
        compiler_params=pltpu.CompilerParams(needs_layout_passes=False),
    )(dst2, zvec)


NB = 2
NI = 4


def _agg_body(g, src1, dst2, zmat, out, srcbuf, acc, r0, r1, d0, d1, d2, d3,
              gs0, gs1, ss0, ss1, is0, is1, is2, is3):
    cid = lax.axis_index("c")
    sid = lax.axis_index("s")
    rows = [r0, r1]
    didx = [d0, d1, d2, d3]
    gsem = [gs0, gs1]
    ssem = [ss0, ss1]
    isem = [is0, is1, is2, is3]

    coff = jnp.where(cid == 0, sid * F0, NS * F0 + sid * F1)
    nch = jnp.where(cid == 0, F0, F1)

    pltpu.sync_copy(zmat, r0)
    for k in range(RC):
        pltpu.sync_copy(r0, acc.at[pl.ds(sid * RT + k * K, K)])

    @pl.when(cid == 0)
    def _():
        pltpu.sync_copy(src1.at[pl.ds(coff * K, F0 * K)], srcbuf.at[pl.ds(0, F0 * K)])

    @pl.when(cid == 1)
    def _():
        pltpu.sync_copy(src1.at[pl.ds(coff * K, F1 * K)], srcbuf.at[pl.ds(0, F1 * K)])

    plsc.subcore_barrier()

    for q in range(NI):
        pltpu.async_copy(dst2.at[coff + q], didx[q], isem[q])
    for b in range(NB):
        pltpu.async_copy(g.at[srcbuf.at[pl.ds(b * K, K)]], rows[b], gsem[b])

    def step(t, carry):
        for cc in range(NI):
            c = t * NI + cc
            b = cc % NB
            q = cc
            pltpu.make_async_copy(
                g.at[srcbuf.at[pl.ds(c * K, K)]], rows[b], gsem[b]
            ).wait()
            pltpu.make_async_copy(dst2.at[coff + c], didx[q], isem[q]).wait()
            pltpu.async_copy(rows[b], acc.at[didx[q]], ssem[b], add=True)
            pltpu.make_async_copy(rows[b], acc.at[didx[q]], ssem[b]).wait()

            @pl.when(c + NI < nch)
            def _():
                pltpu.async_copy(dst2.at[coff + c + NI], didx[q], isem[q])

            @pl.when(c + NB < nch)
            def _():
                pltpu.async_copy(
                    g.at[srcbuf.at[pl.ds((c + NB) * K, K)]], rows[b], gsem[b]
                )

        return carry

    lax.fori_loop(0, nch // NI, step, 0)
    plsc.subcore_barrier()
    for k in range(RC):
        base = sid * RT + k * K
        pltpu.sync_copy(acc.at[pl.ds(base, K)], r0)
        pltpu.sync_copy(r0, out.at[cid, pl.ds(base, K)])


def _agg_call(g, src1, dst2, zmat):
    mesh = plsc.VectorSubcoreMesh(core_axis_name="c", subcore_axis_name="s")
    return pl.kernel(
        _agg_body,
        out_type=jax.ShapeDtypeStruct((NC, P, D), jnp.float32),
        mesh=mesh,
        scratch_types=[
            pltpu.VMEM((F0 * K,), jnp.int32),
            pltpu.VMEM_SHARED((P, D), jnp.float32),
        ]
        + [pltpu.VMEM((K, D), jnp.float32)] * NB
        + [pltpu.VMEM((K,), jnp.int32)] * NI
        + [pltpu.SemaphoreType.DMA] * (NB + NB + NI),
    )(g, src1, dst2, zmat)


def _mm_body(x_ref, w_ref, o_ref):
    o_ref[...] = jnp.dot(x_ref[...], w_ref[...], preferred_element_type=jnp.float32)


def _mm_call(x, w):
    return pl.pallas_call(
        _mm_body,
        grid=(NBLK,),
        in_specs=[
            pl.BlockSpec((BLK, D), lambda i: (i, 0)),
            pl.BlockSpec((D, D), lambda i: (0, 0)),
        ],
        out_specs=pl.BlockSpec((BLK, D), lambda i: (i, 0)),
        out_shape=jax.ShapeDtypeStruct((P, D), jnp.float32),
    )(x, w)


def _scale_body(degt_ref, xw_ref, g_ref):
    deg = jnp.sum(degt_ref[...], axis=1, keepdims=True) + 1.0
    g_ref[...] = xw_ref[...] * lax.rsqrt(deg)


def _scale_call(degt, xw):
    return pl.pallas_call(
        _scale_body,
        grid=(NBLK,),
        in_specs=[
            pl.BlockSpec((BLK, NW), lambda i: (i, 0)),
            pl.BlockSpec((BLK, D), lambda i: (i, 0)),
        ],
        out_specs=pl.BlockSpec((BLK, D), lambda i: (i, 0)),
        out_shape=jax.ShapeDtypeStruct((P, D), jnp.float32),
    )(degt, xw)


def _comb_mm_body(pp_ref, g_ref, degt_ref, b_ref, w_ref, o_ref):
    dv = lax.rsqrt(jnp.sum(degt_ref[...], axis=1, keepdims=True) + 1.0)
    h = dv * (pp_ref[0] + pp_ref[1] + g_ref[...]) + b_ref[...]
    h = jnp.maximum(h, 0.0)
    o_ref[...] = jnp.dot(h, w_ref[...], preferred_element_type=jnp.float32) * dv


def _comb_mm_call(pp, g, degt, b, w):
    return pl.pallas_call(
        _comb_mm_body,
        grid=(NBLK,),
        in_specs=[
            pl.BlockSpec((NC, BLK, D), lambda i: (0, i, 0)),
            pl.BlockSpec((BLK, D), lambda i: (i, 0)),
            pl.BlockSpec((BLK, NW), lambda i: (i, 0)),
            pl.BlockSpec((1, D), lambda i: (0, 0)),
            pl.BlockSpec((D, D), lambda i: (0, 0)),
        ],
        out_specs=pl.BlockSpec((BLK, D), lambda i: (i, 0)),
        out_shape=jax.ShapeDtypeStruct((P, D), jnp.float32),
    )(pp, g, degt, b, w)


def _final_body(qq_ref, g_ref, degt_ref, b_ref, bat_ref, wf_ref, bf_ref,
                o_ref, sums):
    i = pl.program_id(0)

    @pl.when(i == 0)
    def _():
        sums[...] = jnp.zeros_like(sums)

    dv = lax.rsqrt(jnp.sum(degt_ref[...], axis=1, keepdims=True) + 1.0)
    h = dv * (qq_ref[0] + qq_ref[1] + g_ref[...]) + b_ref[...]
    h = jnp.maximum(h, 0.0)
    hx = jnp.concatenate([h, jnp.ones((BLK, D), jnp.float32)], axis=1)
    bt = jnp.broadcast_to(bat_ref[...], (G, BLK))
    rows = lax.broadcasted_iota(jnp.int32, (G, BLK), 0)
    oht = (bt == rows).astype(jnp.float32)
    sums[...] += jnp.dot(oht, hx, preferred_element_type=jnp.float32)

    @pl.when(i == NBLK - 1)
    def _():
        s = sums[...]
        pooled = s[:, :D] / jnp.maximum(s[:, D:], 1.0)
        logits = (
            jnp.dot(pooled, wf_ref[...], preferred_element_type=jnp.float32)
            + bf_ref[...]
        )
        m = jnp.max(logits, axis=1, keepdims=True)
        e = jnp.exp(logits - m)
        o_ref[...] = (e / jnp.sum(e, axis=1, keepdims=True))[:, :NCLS]


def _final_call(qq, g, degt, b, bat, wf, bf):
    return pl.pallas_call(
        _final_body,
        grid=(NBLK,),
        in_specs=[
            pl.BlockSpec((NC, BLK, D), lambda i: (0, i, 0)),
            pl.BlockSpec((BLK, D), lambda i: (i, 0)),
            pl.BlockSpec((BLK, NW), lambda i: (i, 0)),
            pl.BlockSpec((1, D), lambda i: (0, 0)),
            pl.BlockSpec((1, BLK), lambda i: (0, i)),
            pl.BlockSpec((D, D), lambda i: (0, 0)),
            pl.BlockSpec((1, D), lambda i: (0, 0)),
        ],
        out_specs=pl.BlockSpec((G, NCLS), lambda i: (0, 0)),
        out_shape=jax.ShapeDtypeStruct((G, NCLS), jnp.float32),
        scratch_shapes=[pltpu.VMEM((G, 2 * D), jnp.float32)],
    )(qq, g, degt, b, bat, wf, bf)


def kernel(x, edge_index, batch, W1, b1, W2, b2, Wf, bf):
    src = edge_index[0].astype(jnp.int32)
    dst = edge_index[1].astype(jnp.int32)
    npad = EP - E
    pad_ids = N + jnp.arange(npad, dtype=jnp.int32) % (P - N)
    src1 = jnp.concatenate([src, pad_ids])
    dst2 = jnp.concatenate([dst, pad_ids]).reshape(NCHT, K)
    x_p = jnp.pad(x, ((0, P - N), (0, 0)))
    bat1 = jnp.concatenate(
        [batch.astype(jnp.int32), jnp.full((P - N,), G, jnp.int32)]
    ).reshape(1, P)
    zmat = jnp.zeros((K, D), jnp.float32)
    zvec = jnp.zeros((P,), jnp.float32)
    b1r = b1.reshape(1, D)
    b2r = b2.reshape(1, D)
    wfp = jnp.pad(Wf, ((0, 0), (0, D - NCLS)))
    bfp = jnp.concatenate([bf, jnp.full((D - NCLS,), -1e30, jnp.float32)]).reshape(1, D)

    degp = _deg_call(dst2, zvec)
    degt = degp.T
    xw = _mm_call(x_p, W1)
    g1 = _scale_call(degt, xw)
    agg1 = _agg_call(g1, src1, dst2, zmat)
    g2 = _comb_mm_call(agg1, g1, degt, b1r, W2)
    agg2 = _agg_call(g2, src1, dst2, zmat)
    return _final_call(agg2, g2, degt, b2r, bat1, wfp, bfp)


# speedup vs baseline: 3.3528x; 1.0307x over previous
import jax
import jax.numpy as jnp
from jax import lax
from jax.experimental import pallas as pl
from jax.experimental.pallas import tpu as pltpu
from jax.experimental.pallas import tpu_sc as plsc

N = 10000
E = 320000
D = 128
G = 64
NCLS = 10

NC = 2
NS = 16
NW = NC * NS

P = 10240
EW = E // NW
K = 80
NCH = EW // K
NCHT = E // K
RT = P // NS
RC = RT // K

BLK = 512
NBLK = P // BLK


def _deg_body(dst1, zvec, out, dstbuf, degbuf):
    cid = lax.axis_index("c")
    sid = lax.axis_index("s")
    wid = cid * NS + sid
    pltpu.sync_copy(dst1.at[pl.ds(wid * EW, EW)], dstbuf)
    pltpu.sync_copy(zvec, degbuf)
    ones = jnp.ones((16,), jnp.float32)

    def step(i, carry):
        idx = dstbuf[pl.ds(pl.multiple_of(i * 16, 16), 16)]
        plsc.addupdate_scatter(degbuf, [idx], ones)
        return carry

    lax.fori_loop(0, EW // 16, step, 0)
    pltpu.sync_copy(degbuf, out.at[wid])


def _deg_call(dst1, zvec):
    mesh = plsc.VectorSubcoreMesh(core_axis_name="c", subcore_axis_name="s")
    return pl.kernel(
        _deg_body,
        out_type=jax.ShapeDtypeStruct((NW, P), jnp.float32),
        mesh=mesh,
        scratch_types=[
            pltpu.VMEM((EW,), jnp.int32),
            pltpu.VMEM((P,), jnp.float32),
        ],
        compiler_params=pltpu.CompilerParams(needs_layout_passes=False),
    )(dst1, zvec)


NB = 3


def _agg_body(g, src1, dst2, zmat, out, srcbuf, acc, r0, r1, r2, d0, d1, d2,
              gs0, gs1, gs2, ss0, ss1, ss2, is0, is1, is2):
    cid = lax.axis_index("c")
    sid = lax.axis_index("s")
    wid = cid * NS + sid
    coff = wid * NCH
    rows = [r0, r1, r2]
    didx = [d0, d1, d2]
    gsem = [gs0, gs1, gs2]
    ssem = [ss0, ss1, ss2]
    isem = [is0, is1, is2]

    def wait_gather(c, b):
        pltpu.make_async_copy(
            g.at[srcbuf.at[pl.ds(c * K, K)]], rows[b], gsem[b]
        ).wait()

    def issue_scatter(c, b):
        pltpu.make_async_copy(dst2.at[coff + c], didx[b], isem[b]).wait()
        pltpu.async_copy(rows[b], acc.at[didx[b]], ssem[b], add=True)

    def drain_scatter(c, b):
        pltpu.make_async_copy(rows[b], acc.at[didx[b]], ssem[b]).wait()

    def refill(c, b):
        pltpu.async_copy(dst2.at[coff + c], didx[b], isem[b])
        pltpu.async_copy(g.at[srcbuf.at[pl.ds(c * K, K)]], rows[b], gsem[b])

    pltpu.sync_copy(zmat, r0)
    for k in range(RC):
        pltpu.sync_copy(r0, acc.at[pl.ds(sid * RT + k * K, K)])
    pltpu.sync_copy(src1.at[pl.ds(wid * EW, EW)], srcbuf)
    plsc.subcore_barrier()

    for b in range(NB):
        pltpu.async_copy(dst2.at[coff + b], didx[b], isem[b])
        pltpu.async_copy(g.at[srcbuf.at[pl.ds(b * K, K)]], rows[b], gsem[b])

    wait_gather(0, 0)
    issue_scatter(0, 0)

    def step(t, carry):
        for cc in range(NB):
            c = 1 + t * NB + cc
            b = (1 + cc) % NB
            bp = cc % NB
            wait_gather(c, b)
            issue_scatter(c, b)
            drain_scatter(c - 1, bp)

            @pl.when(c + 2 < NCH)
            def _():
                refill(c + 2, bp)

        return carry

    lax.fori_loop(0, (NCH - 2) // NB, step, 0)

    wait_gather(NCH - 1, (NCH - 1) % NB)
    issue_scatter(NCH - 1, (NCH - 1) % NB)
    drain_scatter(NCH - 2, (NCH - 2) % NB)
    drain_scatter(NCH - 1, (NCH - 1) % NB)

    plsc.subcore_barrier()
    for k in range(RC):
        base = sid * RT + k * K
        pltpu.sync_copy(acc.at[pl.ds(base, K)], r0)
        pltpu.sync_copy(r0, out.at[cid, pl.ds(base, K)])


def _agg_call(g, src1, dst2, zmat):
    mesh = plsc.VectorSubcoreMesh(core_axis_name="c", subcore_axis_name="s")
    return pl.kernel(
        _agg_body,
        out_type=jax.ShapeDtypeStruct((NC, P, D), jnp.float32),
        mesh=mesh,
        scratch_types=[
            pltpu.VMEM((EW,), jnp.int32),
            pltpu.VMEM_SHARED((P, D), jnp.float32),
        ]
        + [pltpu.VMEM((K, D), jnp.float32)] * NB
        + [pltpu.VMEM((K,), jnp.int32)] * NB
        + [pltpu.SemaphoreType.DMA] * (3 * NB),
    )(g, src1, dst2, zmat)


def _mm_body(x_ref, w_ref, o_ref):
    o_ref[...] = jnp.dot(x_ref[...], w_ref[...], preferred_element_type=jnp.float32)


def _mm_call(x, w):
    return pl.pallas_call(
        _mm_body,
        grid=(NBLK,),
        in_specs=[
            pl.BlockSpec((BLK, D), lambda i: (i, 0)),
            pl.BlockSpec((D, D), lambda i: (0, 0)),
        ],
        out_specs=pl.BlockSpec((BLK, D), lambda i: (i, 0)),
        out_shape=jax.ShapeDtypeStruct((P, D), jnp.float32),
    )(x, w)


def _scale_body(degt_ref, xw_ref, g_ref):
    deg = jnp.sum(degt_ref[...], axis=1, keepdims=True) + 1.0
    g_ref[...] = xw_ref[...] * lax.rsqrt(deg)


def _scale_call(degt, xw):
    return pl.pallas_call(
        _scale_body,
        grid=(NBLK,),
        in_specs=[
            pl.BlockSpec((BLK, NW), lambda i: (i, 0)),
            pl.BlockSpec((BLK, D), lambda i: (i, 0)),
        ],
        out_specs=pl.BlockSpec((BLK, D), lambda i: (i, 0)),
        out_shape=jax.ShapeDtypeStruct((P, D), jnp.float32),
    )(degt, xw)


def _comb_mm_body(pp_ref, g_ref, degt_ref, b_ref, w_ref, o_ref):
    dv = lax.rsqrt(jnp.sum(degt_ref[...], axis=1, keepdims=True) + 1.0)
    h = dv * (pp_ref[0] + pp_ref[1] + g_ref[...]) + b_ref[...]
    h = jnp.maximum(h, 0.0)
    o_ref[...] = jnp.dot(h, w_ref[...], preferred_element_type=jnp.float32) * dv


def _comb_mm_call(pp, g, degt, b, w):
    return pl.pallas_call(
        _comb_mm_body,
        grid=(NBLK,),
        in_specs=[
            pl.BlockSpec((NC, BLK, D), lambda i: (0, i, 0)),
            pl.BlockSpec((BLK, D), lambda i: (i, 0)),
            pl.BlockSpec((BLK, NW), lambda i: (i, 0)),
            pl.BlockSpec((1, D), lambda i: (0, 0)),
            pl.BlockSpec((D, D), lambda i: (0, 0)),
        ],
        out_specs=pl.BlockSpec((BLK, D), lambda i: (i, 0)),
        out_shape=jax.ShapeDtypeStruct((P, D), jnp.float32),
    )(pp, g, degt, b, w)


def _final_body(qq_ref, g_ref, degt_ref, b_ref, bat_ref, wf_ref, bf_ref,
                o_ref, sums):
    i = pl.program_id(0)

    @pl.when(i == 0)
    def _():
        sums[...] = jnp.zeros_like(sums)

    dv = lax.rsqrt(jnp.sum(degt_ref[...], axis=1, keepdims=True) + 1.0)
    h = dv * (qq_ref[0] + qq_ref[1] + g_ref[...]) + b_ref[...]
    h = jnp.maximum(h, 0.0)
    hx = jnp.concatenate([h, jnp.ones((BLK, D), jnp.float32)], axis=1)
    bt = jnp.broadcast_to(bat_ref[...], (G, BLK))
    rows = lax.broadcasted_iota(jnp.int32, (G, BLK), 0)
    oht = (bt == rows).astype(jnp.float32)
    sums[...] += jnp.dot(oht, hx, preferred_element_type=jnp.float32)

    @pl.when(i == NBLK - 1)
    def _():
        s = sums[...]
        pooled = s[:, :D] / jnp.maximum(s[:, D:], 1.0)
        logits = (
            jnp.dot(pooled, wf_ref[...], preferred_element_type=jnp.float32)
            + bf_ref[...]
        )
        m = jnp.max(logits, axis=1, keepdims=True)
        e = jnp.exp(logits - m)
        o_ref[...] = (e / jnp.sum(e, axis=1, keepdims=True))[:, :NCLS]


def _final_call(qq, g, degt, b, bat, wf, bf):
    return pl.pallas_call(
        _final_body,
        grid=(NBLK,),
        in_specs=[
            pl.BlockSpec((NC, BLK, D), lambda i: (0, i, 0)),
            pl.BlockSpec((BLK, D), lambda i: (i, 0)),
            pl.BlockSpec((BLK, NW), lambda i: (i, 0)),
            pl.BlockSpec((1, D), lambda i: (0, 0)),
            pl.BlockSpec((1, BLK), lambda i: (0, i)),
            pl.BlockSpec((D, D), lambda i: (0, 0)),
            pl.BlockSpec((1, D), lambda i: (0, 0)),
        ],
        out_specs=pl.BlockSpec((G, NCLS), lambda i: (0, 0)),
        out_shape=jax.ShapeDtypeStruct((G, NCLS), jnp.float32),
        scratch_shapes=[pltpu.VMEM((G, 2 * D), jnp.float32)],
    )(qq, g, degt, b, bat, wf, bf)


def kernel(x, edge_index, batch, W1, b1, W2, b2, Wf, bf):
    src1 = edge_index[0].astype(jnp.int32)
    dst1 = edge_index[1].astype(jnp.int32)
    dst2 = dst1.reshape(NCHT, K)
    x_p = jnp.pad(x, ((0, P - N), (0, 0)))
    bat1 = jnp.concatenate(
        [batch.astype(jnp.int32), jnp.full((P - N,), G, jnp.int32)]
    ).reshape(1, P)
    zmat = jnp.zeros((K, D), jnp.float32)
    zvec = jnp.zeros((P,), jnp.float32)
    b1r = b1.reshape(1, D)
    b2r = b2.reshape(1, D)
    wfp = jnp.pad(Wf, ((0, 0), (0, D - NCLS)))
    bfp = jnp.concatenate([bf, jnp.full((D - NCLS,), -1e30, jnp.float32)]).reshape(1, D)

    degp = _deg_call(dst1, zvec)
    degt = degp.T
    xw = _mm_call(x_p, W1)
    g1 = _scale_call(degt, xw)
    agg1 = _agg_call(g1, src1, dst2, zmat)
    g2 = _comb_mm_call(agg1, g1, degt, b1r, W2)
    agg2 = _agg_call(g2, src1, dst2, zmat)
    return _final_call(agg2, g2, degt, b2r, bat1, wfp, bfp)
